# Initial kernel scaffold; baseline (speedup 1.0000x reference)
import jax
import jax.numpy as jnp
from jax.experimental import pallas as pl


def kernel(H_tilde, S, edge_index, time_idx, W_q, W_k, W_v):
    raise NotImplementedError("write your pallas kernel here")


# speedup vs baseline: 14.4541x; 14.4541x over previous
import math

import jax
import jax.numpy as jnp
from jax.experimental import pallas as pl
from jax.experimental.pallas import tpu as pltpu

_T = 8
_N = 512
_DIN = 256
_D = 256
_H = 8
_DH = 32
_W = 4
_WP = 8
_TOPK = 16
_TEMP = 1.0
_NEG = -1e9
_TAUS = (4.0, 16.0)
_NFREQ = 3
_SCALE = _DH ** (-0.5)


def _proj_kernel(ht_ref, wq_ref, wk_ref, wv_ref, q_ref, k_ref, v_ref):
    x = ht_ref[0]
    for w_ref, o_ref in ((wq_ref, q_ref), (wk_ref, k_ref), (wv_ref, v_ref)):
        r = jax.lax.dot_general(w_ref[...], x, (((1,), (0,)), ((), ())),
                                preferred_element_type=jnp.float32)
        o_ref[:, 0] = r.reshape(_H, _DH, _N)


def _attn_kernel(qt_ref, kt_ref, vt_ref, a_ref, s_ref, pe_ref, o_ref,
                 sc_ref, act_ref):
    t = pl.program_id(0)
    amask = a_ref[...] > 0.0

    def head_body(h, carry):
        qT = qt_ref[h, 0]
        for dt in range(_W + 1):
            tpc = jnp.maximum(t - dt, 0)
            kT = kt_ref[h, tpc] + pe_ref[h, dt][:, None]
            s = jax.lax.dot_general(qT, kT, (((0,), (0,)), ((), ())),
                                    preferred_element_type=jnp.float32)
            gate = jnp.log(jnp.clip(s_ref[tpc], 0.0, 1.0) + 1e-6)
            s = s * _SCALE + (-math.log1p(float(dt))) + gate[None, :]
            valid = jnp.logical_and(amask, dt <= t)
            sc_ref[:, dt * _N:(dt + 1) * _N] = jnp.where(valid, s / _TEMP, _NEG)

        act_ref[...] = sc_ref[...]

        def peel(_, c):
            m = jnp.max(act_ref[...], axis=1, keepdims=True)
            act_ref[...] = jnp.where(act_ref[...] == m, _NEG, act_ref[...])
            return c

        jax.lax.fori_loop(0, _TOPK - 1, peel, 0)
        thresh = jnp.max(act_ref[...], axis=1, keepdims=True)
        m_row = jnp.max(sc_ref[...], axis=1, keepdims=True)

        pv = jnp.zeros((_N, _DH), jnp.float32)
        denom = jnp.zeros((_N, 1), jnp.float32)
        for dt in range(_W + 1):
            tpc = jnp.maximum(t - dt, 0)
            l_c = sc_ref[:, dt * _N:(dt + 1) * _N]
            keep = jnp.logical_and(l_c >= thresh,
                                   jnp.logical_and(amask, dt <= t))
            e = jnp.where(keep, jnp.exp(l_c - m_row), 0.0)
            denom = denom + jnp.sum(e, axis=1, keepdims=True)
            pv = pv + jax.lax.dot_general(e, vt_ref[h, tpc],
                                          (((1,), (1,)), ((), ())),
                                          preferred_element_type=jnp.float32)
        o_ref[h, 0] = pv / jnp.maximum(denom, 1e-12)
        return carry

    jax.lax.fori_loop(0, _H, head_body, 0)


def kernel(H_tilde, S, edge_index, time_idx, W_q, W_k, W_v):
    A = jnp.zeros((_N, _N), jnp.float32).at[edge_index[1], edge_index[0]].set(1.0)

    dts = jnp.arange(_W + 1, dtype=jnp.float32)
    decays = jnp.stack([jnp.exp(-dts / tau) for tau in _TAUS], axis=-1)
    freqs = 1.0 / (10000.0 ** (jnp.arange(_NFREQ, dtype=jnp.float32) / _NFREQ))
    ang = dts[:, None] * freqs[None, :]
    pe_table = jnp.concatenate([decays, jnp.sin(ang), jnp.cos(ang)], axis=-1)
    pe_proj = pe_table @ W_k[_DIN:]
    peT = jnp.pad(pe_proj, ((0, _WP - (_W + 1)), (0, 0)))
    peT = peT.reshape(_WP, _H, _DH).transpose(1, 0, 2)

    HT = jnp.transpose(H_tilde, (0, 2, 1))
    WqT = W_q.T
    WkT = W_k[:_DIN].T
    WvT = W_v.T

    qt, kt, vt = pl.pallas_call(
        _proj_kernel,
        grid=(_T,),
        in_specs=[
            pl.BlockSpec((1, _DIN, _N), lambda t: (t, 0, 0)),
            pl.BlockSpec((_D, _DIN), lambda t: (0, 0)),
            pl.BlockSpec((_D, _DIN), lambda t: (0, 0)),
            pl.BlockSpec((_D, _DIN), lambda t: (0, 0)),
        ],
        out_specs=[
            pl.BlockSpec((_H, 1, _DH, _N), lambda t: (0, t, 0, 0)),
            pl.BlockSpec((_H, 1, _DH, _N), lambda t: (0, t, 0, 0)),
            pl.BlockSpec((_H, 1, _DH, _N), lambda t: (0, t, 0, 0)),
        ],
        out_shape=[jax.ShapeDtypeStruct((_H, _T, _DH, _N), jnp.float32)] * 3,
    )(HT, WqT, WkT, WvT)

    oh = pl.pallas_call(
        _attn_kernel,
        grid=(_T,),
        in_specs=[
            pl.BlockSpec((_H, 1, _DH, _N), lambda t: (0, t, 0, 0)),
            pl.BlockSpec((_H, _T, _DH, _N), lambda t: (0, 0, 0, 0)),
            pl.BlockSpec((_H, _T, _DH, _N), lambda t: (0, 0, 0, 0)),
            pl.BlockSpec((_N, _N), lambda t: (0, 0)),
            pl.BlockSpec((_T, _N), lambda t: (0, 0)),
            pl.BlockSpec((_H, _WP, _DH), lambda t: (0, 0, 0)),
        ],
        out_specs=pl.BlockSpec((_H, 1, _N, _DH), lambda t: (0, t, 0, 0)),
        out_shape=jax.ShapeDtypeStruct((_H, _T, _N, _DH), jnp.float32),
        scratch_shapes=[
            pltpu.VMEM((_N, (_W + 1) * _N), jnp.float32),
            pltpu.VMEM((_N, (_W + 1) * _N), jnp.float32),
        ],
    )(qt, kt, vt, A, S, peT)

    return oh.transpose(1, 2, 0, 3).reshape(_T, _N, _D)
